# baseline (speedup 1.0000x reference)
import jax
import jax.numpy as jnp
from jax import lax
from jax.experimental import pallas as pl
from jax.experimental.pallas import tpu as pltpu, tpu_sc as plsc

SIZE = 16777216
GROUP_SIZE = 4
TEMP_INIT = 1.0
TEMP_FINAL = 0.1
ANNEAL_STEPS = 10000

CHUNK = 8192
NBUF = 4
TILE = 512
LANES = 16


def _temperature(step):
    step_f = jnp.maximum(jnp.asarray(step), 0).astype(jnp.float32)
    frac = jnp.minimum(jnp.float32(1.0), step_f / jnp.float32(ANNEAL_STEPS))
    t = jnp.float32(TEMP_INIT) + frac * (jnp.float32(TEMP_FINAL) - jnp.float32(TEMP_INIT))
    return jnp.maximum(t, jnp.float32(1e-06))


def _compute_chunk(in_v, out_v, tv):
    qi = 4 * lax.iota(jnp.int32, LANES)
    one = jnp.full((LANES,), 1.0, dtype=jnp.float32)
    zero = jnp.full((LANES,), 0.0, dtype=jnp.float32)

    def maj(a, b, c):
        return (a & b) | (c & (a | b))

    step64 = jnp.full((LANES,), 64, dtype=jnp.int32)
    carry0 = (qi, qi + 1, qi + 2, qi + 3)

    @plsc.parallel_loop(0, CHUNK // (4 * LANES), unroll=8, carry=carry0)
    def blk(i, c):
        i0, i1, i2, i3 = c
        off = (i // 8) * TILE + (i % 8) * LANES
        v0 = in_v[pl.ds(off, LANES)] / tv
        v1 = in_v[pl.ds(off + 128, LANES)] / tv
        v2 = in_v[pl.ds(off + 256, LANES)] / tv
        v3 = in_v[pl.ds(off + 384, LANES)] / tv
        b01 = v0 >= v1
        b02 = v0 >= v2
        b03 = v0 >= v3
        b12 = v1 >= v2
        b13 = v1 >= v3
        b23 = v2 >= v3
        m0 = maj(b01, b02, b03)
        m1 = maj(~b01, b12, b13)
        m2 = maj(~b02, ~b12, b23)
        m3 = ~maj(b03, b13, b23)
        plsc.store_scatter(out_v, [i0], jnp.where(m0, one, zero))
        plsc.store_scatter(out_v, [i1], jnp.where(m1, one, zero))
        plsc.store_scatter(out_v, [i2], jnp.where(m2, one, zero))
        plsc.store_scatter(out_v, [i3], jnp.where(m3, one, zero))
        return (i0 + step64, i1 + step64, i2 + step64, i3 + step64)


def _sc_topk_mask(flat, tvec):
    info = plsc.get_sparse_core_info()
    nc, ns = info.num_cores, info.num_subcores
    nw = nc * ns
    per_w = SIZE // nw
    nch = per_w // CHUNK
    pairs = nch // 2
    mesh = plsc.VectorSubcoreMesh(core_axis_name="c", subcore_axis_name="s")

    quads = nch // NBUF

    def body(x_hbm, t_hbm, out_hbm,
             ib0, ib1, ib2, ib3, ob0, ob1, ob2, ob3, t_v,
             si0, si1, si2, si3, so0, so1, so2, so3):
        inb = (ib0, ib1, ib2, ib3)
        outb = (ob0, ob1, ob2, ob3)
        sin = (si0, si1, si2, si3)
        sout = (so0, so1, so2, so3)
        wid = lax.axis_index("s") * nc + lax.axis_index("c")
        base = wid * per_w
        pltpu.sync_copy(t_hbm, t_v)
        tv = t_v[...]

        def in_slice(g):
            return x_hbm.at[pl.ds(base + g * CHUNK, CHUNK)]

        def out_slice(g):
            return out_hbm.at[pl.ds(base + g * CHUNK, CHUNK)]

        for b in range(NBUF):
            pltpu.async_copy(in_slice(b), inb[b], sin[b])

        for b in range(NBUF):
            pltpu.make_async_copy(in_slice(b), inb[b], sin[b]).wait()
            _compute_chunk(inb[b], outb[b], tv)
            pltpu.async_copy(outb[b], out_slice(b), sout[b])
            pltpu.async_copy(in_slice(NBUF + b), inb[b], sin[b])

        def quad(q, carry):
            g0 = NBUF * q
            for b in range(NBUF):
                pltpu.make_async_copy(in_slice(g0 + b), inb[b], sin[b]).wait()
                pltpu.make_async_copy(outb[b], out_slice(g0 + b), sout[b]).wait()
                _compute_chunk(inb[b], outb[b], tv)
                pltpu.async_copy(outb[b], out_slice(g0 + b), sout[b])
                pltpu.async_copy(in_slice(g0 + NBUF + b), inb[b], sin[b])
            return carry

        lax.fori_loop(1, quads - 1, quad, 0)

        g0 = NBUF * (quads - 1)
        for b in range(NBUF):
            pltpu.make_async_copy(in_slice(g0 + b), inb[b], sin[b]).wait()
            pltpu.make_async_copy(outb[b], out_slice(g0 + b), sout[b]).wait()
            _compute_chunk(inb[b], outb[b], tv)
            pltpu.async_copy(outb[b], out_slice(g0 + b), sout[b])
        for b in range(NBUF):
            pltpu.make_async_copy(outb[b], out_slice(g0 + b), sout[b]).wait()

    call = pl.kernel(
        body,
        out_type=jax.ShapeDtypeStruct((SIZE,), jnp.float32),
        mesh=mesh,
        compiler_params=pltpu.CompilerParams(needs_layout_passes=False),
        scratch_types=(
            [pltpu.VMEM((CHUNK,), jnp.float32)] * (2 * NBUF)
            + [pltpu.VMEM((LANES,), jnp.float32)]
            + [pltpu.SemaphoreType.DMA] * (2 * NBUF)
        ),
    )
    return call(flat, tvec)


def kernel(logits, step=0, sample=0):
    x = logits.reshape(SIZE // (4 * 128), 128, 4)
    x = jnp.swapaxes(x, 1, 2)
    flat = x.reshape(-1)
    tvec = jnp.full((LANES,), _temperature(step), dtype=jnp.float32)
    return _sc_topk_mask(flat, tvec)


# speedup vs baseline: 1.0170x; 1.0170x over previous
import jax
import jax.numpy as jnp
from jax import lax
from jax.experimental import pallas as pl
from jax.experimental.pallas import tpu as pltpu, tpu_sc as plsc

SIZE = 16777216
GROUP_SIZE = 4
TEMP_INIT = 1.0
TEMP_FINAL = 0.1
ANNEAL_STEPS = 10000

CHUNK = 16384
NBUF = 2
TILE = 512
LANES = 16


def _temperature(step):
    step_f = jnp.maximum(jnp.asarray(step), 0).astype(jnp.float32)
    frac = jnp.minimum(jnp.float32(1.0), step_f / jnp.float32(ANNEAL_STEPS))
    t = jnp.float32(TEMP_INIT) + frac * (jnp.float32(TEMP_FINAL) - jnp.float32(TEMP_INIT))
    return jnp.maximum(t, jnp.float32(1e-06))


def _compute_chunk(in_v, out_v, tv):
    qi = 4 * lax.iota(jnp.int32, LANES)
    one = jnp.full((LANES,), 1.0, dtype=jnp.float32)
    zero = jnp.full((LANES,), 0.0, dtype=jnp.float32)

    def maj(a, b, c):
        return (a & b) | (c & (a | b))

    step64 = jnp.full((LANES,), 64, dtype=jnp.int32)
    carry0 = (qi, qi + 1, qi + 2, qi + 3)

    @plsc.parallel_loop(0, CHUNK // (4 * LANES), unroll=8, carry=carry0)
    def blk(i, c):
        i0, i1, i2, i3 = c
        off = (i // 8) * TILE + (i % 8) * LANES
        v0 = in_v[pl.ds(off, LANES)] / tv
        v1 = in_v[pl.ds(off + 128, LANES)] / tv
        v2 = in_v[pl.ds(off + 256, LANES)] / tv
        v3 = in_v[pl.ds(off + 384, LANES)] / tv
        b01 = v0 >= v1
        b02 = v0 >= v2
        b03 = v0 >= v3
        b12 = v1 >= v2
        b13 = v1 >= v3
        b23 = v2 >= v3
        m0 = maj(b01, b02, b03)
        m1 = maj(~b01, b12, b13)
        m2 = maj(~b02, ~b12, b23)
        m3 = ~maj(b03, b13, b23)
        plsc.store_scatter(out_v, [i0], jnp.where(m0, one, zero))
        plsc.store_scatter(out_v, [i1], jnp.where(m1, one, zero))
        plsc.store_scatter(out_v, [i2], jnp.where(m2, one, zero))
        plsc.store_scatter(out_v, [i3], jnp.where(m3, one, zero))
        return (i0 + step64, i1 + step64, i2 + step64, i3 + step64)


def _sc_topk_mask(flat, tvec):
    info = plsc.get_sparse_core_info()
    nc, ns = info.num_cores, info.num_subcores
    nw = nc * ns
    per_w = SIZE // nw
    nch = per_w // CHUNK
    pairs = nch // 2
    mesh = plsc.VectorSubcoreMesh(core_axis_name="c", subcore_axis_name="s")

    quads = nch // NBUF

    def body(*refs):
        x_hbm, t_hbm, out_hbm = refs[:3]
        inb = refs[3:3 + NBUF]
        outb = refs[3 + NBUF:3 + 2 * NBUF]
        t_v = refs[3 + 2 * NBUF]
        sems = refs[4 + 2 * NBUF:]
        sin = sems[:NBUF]
        sout = sems[NBUF:]
        wid = lax.axis_index("s") * nc + lax.axis_index("c")
        base = wid * per_w
        pltpu.sync_copy(t_hbm, t_v)
        tv = t_v[...]

        def in_slice(g):
            return x_hbm.at[pl.ds(base + g * CHUNK, CHUNK)]

        def out_slice(g):
            return out_hbm.at[pl.ds(base + g * CHUNK, CHUNK)]

        for b in range(NBUF):
            pltpu.async_copy(in_slice(b), inb[b], sin[b])

        for b in range(NBUF):
            pltpu.make_async_copy(in_slice(b), inb[b], sin[b]).wait()
            _compute_chunk(inb[b], outb[b], tv)
            pltpu.async_copy(outb[b], out_slice(b), sout[b])
            pltpu.async_copy(in_slice(NBUF + b), inb[b], sin[b])

        def quad(q, carry):
            g0 = NBUF * q
            for b in range(NBUF):
                pltpu.make_async_copy(in_slice(g0 + b), inb[b], sin[b]).wait()
                pltpu.make_async_copy(outb[b], out_slice(g0 + b), sout[b]).wait()
                _compute_chunk(inb[b], outb[b], tv)
                pltpu.async_copy(outb[b], out_slice(g0 + b), sout[b])
                pltpu.async_copy(in_slice(g0 + NBUF + b), inb[b], sin[b])
            return carry

        lax.fori_loop(1, quads - 1, quad, 0)

        g0 = NBUF * (quads - 1)
        for b in range(NBUF):
            pltpu.make_async_copy(in_slice(g0 + b), inb[b], sin[b]).wait()
            pltpu.make_async_copy(outb[b], out_slice(g0 + b), sout[b]).wait()
            _compute_chunk(inb[b], outb[b], tv)
            pltpu.async_copy(outb[b], out_slice(g0 + b), sout[b])
        for b in range(NBUF):
            pltpu.make_async_copy(outb[b], out_slice(g0 + b), sout[b]).wait()

    call = pl.kernel(
        body,
        out_type=jax.ShapeDtypeStruct((SIZE,), jnp.float32),
        mesh=mesh,
        compiler_params=pltpu.CompilerParams(needs_layout_passes=False),
        scratch_types=(
            [pltpu.VMEM((CHUNK,), jnp.float32)] * (2 * NBUF)
            + [pltpu.VMEM((LANES,), jnp.float32)]
            + [pltpu.SemaphoreType.DMA] * (2 * NBUF)
        ),
    )
    return call(flat, tvec)


def kernel(logits, step=0, sample=0):
    x = logits.reshape(SIZE // (4 * 128), 128, 4)
    x = jnp.swapaxes(x, 1, 2)
    flat = x.reshape(-1)
    tvec = jnp.full((LANES,), _temperature(step), dtype=jnp.float32)
    return _sc_topk_mask(flat, tvec)
